# Initial kernel scaffold; baseline (speedup 1.0000x reference)
import jax
import jax.numpy as jnp
from jax.experimental import pallas as pl


def kernel(x, emb_linear, emb_table, bias, W1, b1, W2, b2, W3, b3):
    raise NotImplementedError("write your pallas kernel here")


# speedup vs baseline: 1.0480x; 1.0480x over previous
import functools

import jax
import jax.numpy as jnp
from jax import lax
from jax.experimental import pallas as pl
from jax.experimental.pallas import tpu as pltpu
from jax.experimental.pallas import tpu_sc as plsc

V = 1000000
D = 64
B = 16384
F = 26

NC = 2
NS = 16
NW = NC * NS
SPW = B // NW
CHUNK = 4
G = CHUNK * F
NCH = SPW // CHUNK
DL = D // 16


def _sc_body(idx_hbm, table_hbm, lin_hbm, out_inter, out_lin,
             idx_v, buf_a, buf_b, out_v, lin_acc, sem_a, sem_b, sem_l):
    wid = lax.axis_index("s") * NC + lax.axis_index("c")

    pltpu.sync_copy(idx_hbm.at[wid], idx_v)

    def fire_rows(j, buf, sem):
        pltpu.make_async_copy(table_hbm.at[idx_v.at[j]], buf, sem).start()

    def wait_rows(j, buf, sem):
        pltpu.make_async_copy(table_hbm.at[idx_v.at[j]], buf, sem).wait()

    def fire_lin(j):
        pltpu.make_async_copy(lin_hbm.at[idx_v.at[j]], lin_acc.at[j], sem_l).start()

    def wait_lin(j):
        pltpu.make_async_copy(lin_hbm.at[idx_v.at[j]], lin_acc.at[j], sem_l).wait()

    fire_rows(0, buf_a, sem_a)
    fire_lin(0)
    fire_rows(1, buf_b, sem_b)
    fire_lin(1)

    def step(t, carry):
        for slot, (buf, sem) in enumerate(((buf_a, sem_a), (buf_b, sem_b))):
            j = 2 * t + slot
            wait_rows(j, buf, sem)
            wait_lin(j)
            for s in range(CHUNK):
                acc = [jnp.zeros((16,), jnp.float32) for _ in range(DL)]
                accq = [jnp.zeros((16,), jnp.float32) for _ in range(DL)]
                for r in range(F):
                    row = s * F + r
                    for c in range(DL):
                        v = buf[row, pl.ds(c * 16, 16)]
                        acc[c] = acc[c] + v
                        accq[c] = accq[c] + v * v
                orow = CHUNK * j + s
                for c in range(DL):
                    out_v[orow, pl.ds(c * 16, 16)] = 0.5 * (
                        acc[c] * acc[c] - accq[c])
            nxt = j + 2

            @pl.when(nxt < NCH)
            def _():
                fire_rows(nxt, buf, sem)
                fire_lin(nxt)
        return carry

    lax.fori_loop(0, NCH // 2, step, 0)

    pltpu.sync_copy(out_v, out_inter.at[pl.ds(wid * SPW, SPW)])
    pltpu.sync_copy(lin_acc, out_lin.at[wid])


@jax.jit
def _sc_gather(idx, table, lin_tab):
    mesh = plsc.VectorSubcoreMesh(core_axis_name="c", subcore_axis_name="s")
    f = pl.kernel(
        _sc_body,
        mesh=mesh,
        compiler_params=pltpu.CompilerParams(use_tc_tiling_on_sc=False),
        out_type=[
            jax.ShapeDtypeStruct((B, D), jnp.float32),
            jax.ShapeDtypeStruct((NW, NCH, G), jnp.float32),
        ],
        scratch_types=[
            pltpu.VMEM((NCH, G), jnp.int32),
            pltpu.VMEM((G, D), jnp.float32),
            pltpu.VMEM((G, D), jnp.float32),
            pltpu.VMEM((SPW, D), jnp.float32),
            pltpu.VMEM((NCH, G), jnp.float32),
            pltpu.SemaphoreType.DMA,
            pltpu.SemaphoreType.DMA,
            pltpu.SemaphoreType.DMA,
        ],
    )
    return f(idx, table, lin_tab)


def _mlp_body(inter_ref, lin_ref, w1_ref, b1_ref, w2_ref, b2_ref, w3_ref,
              c_ref, out_ref):
    inter = inter_ref[...]
    h = jnp.dot(inter, w1_ref[...], preferred_element_type=jnp.float32)
    h = jnp.maximum(h + b1_ref[...], 0.0)
    h = jnp.dot(h, w2_ref[...], preferred_element_type=jnp.float32)
    h = jnp.maximum(h + b2_ref[...], 0.0)
    deep = jnp.sum(h * w3_ref[...], axis=1, keepdims=True)
    lr = jnp.sum(lin_ref[...], axis=1, keepdims=True)
    out_ref[...] = jax.nn.sigmoid(deep + lr + c_ref[...])


@jax.jit
def _mlp(inter, lin2, W1, b1r, W2, b2r, w3r, c):
    blk = 2048
    grid = (B // blk,)
    return pl.pallas_call(
        _mlp_body,
        grid=grid,
        in_specs=[
            pl.BlockSpec((blk, D), lambda i: (i, 0)),
            pl.BlockSpec((blk, F), lambda i: (i, 0)),
            pl.BlockSpec((D, 256), lambda i: (0, 0)),
            pl.BlockSpec((1, 256), lambda i: (0, 0)),
            pl.BlockSpec((256, 128), lambda i: (0, 0)),
            pl.BlockSpec((1, 128), lambda i: (0, 0)),
            pl.BlockSpec((1, 128), lambda i: (0, 0)),
            pl.BlockSpec((1, 1), lambda i: (0, 0)),
        ],
        out_specs=pl.BlockSpec((blk, 1), lambda i: (i, 0)),
        out_shape=jax.ShapeDtypeStruct((B, 1), jnp.float32),
    )(inter, lin2, W1, b1r, W2, b2r, w3r, c)


def kernel(x, emb_linear, emb_table, bias, W1, b1, W2, b2, W3, b3):
    idx = x.astype(jnp.int32).reshape(NW, NCH, G)
    lin_tab = emb_linear.reshape(V)
    inter, lin_vals = _sc_gather(idx, emb_table, lin_tab)
    lin2 = lin_vals.reshape(B, F)
    c = (b3 + bias).reshape(1, 1)
    return _mlp(inter, lin2, W1, b1.reshape(1, 256), W2, b2.reshape(1, 128),
                W3.reshape(1, 128), c)
